# baseline (speedup 1.0000x reference)
import functools

import jax
import jax.numpy as jnp
from jax import lax
from jax.experimental import pallas as pl
from jax.experimental.pallas import tpu as pltpu
from jax.experimental.pallas import tpu_sc as plsc

_B, _L = 4096, 200
_D = 32
_N = _B * _L

_NC, _NS = 2, 16
_NW = _NC * _NS
_RW = _B // _NW
_CROWS = 16


def _area_body(x0_ref, x1_ref, wt_ref, bt_ref, o_ref):
    x0 = x0_ref[...]
    x1 = x1_ref[...]
    wt = wt_ref[...]
    bt = bt_ref[...]
    w0 = wt[:, 0:1][None]
    w1 = wt[:, 1:2][None]
    y = x0[:, None, :] * w0 + x1[:, None, :] * w1 + bt[None]
    o_ref[...] = jnp.maximum(y, 0.0)


_AREA_LB = 8

_area_call = pl.pallas_call(
    _area_body,
    out_shape=jax.ShapeDtypeStruct((_L, _D, _B), jnp.float32),
    grid=(_L // _AREA_LB,),
    in_specs=[
        pl.BlockSpec((_AREA_LB, _B), lambda i: (i, 0)),
        pl.BlockSpec((_AREA_LB, _B), lambda i: (i, 0)),
        pl.BlockSpec((_D, 2), lambda i: (0, 0)),
        pl.BlockSpec((_D, 1), lambda i: (0, 0)),
    ],
    out_specs=pl.BlockSpec((_AREA_LB, _D, _B), lambda i: (i, 0, 0)),
)


_BT = 128
_LANES = 16


_TP = _BT + 1


def _transpose_128x32(grows, trows):
    rows0 = lax.iota(jnp.int32, _LANES)
    rows1 = rows0 + _LANES
    for b0 in range(0, _BT, 4):
        vals = []
        for b in range(b0, b0 + 4):
            col = jnp.full((_LANES,), b, jnp.int32)
            for dh, rows in ((0, rows0), (1, rows1)):
                x = grows[b, pl.ds(dh * _LANES, _LANES)]
                vals.append((rows, col, x))
        for rows, col, x in vals:
            plsc.store_scatter(trows, [rows, col], x)


_NB = 4


def _gather_body(shotT_hbm, playerT_hbm, stab_hbm, ptab_hbm, out_s, out_p,
                 idx_v, g0, g1, g2, g3, t0, t1, t2, t3,
                 gs0, gs1, gs2, gs3, ws0, ws1, ws2, ws3):
    wid = lax.axis_index("s") * _NC + lax.axis_index("c")
    gbufs = ((g0, gs0), (g1, gs1), (g2, gs2), (g3, gs3))
    tbufs = ((t0, ws0), (t1, ws1), (t2, ws2), (t3, ws3))
    for tab, idxT_hbm, out in ((stab_hbm, shotT_hbm, out_s),
                               (ptab_hbm, playerT_hbm, out_p)):
        pltpu.sync_copy(idxT_hbm.at[:, pl.ds(wid * _BT, _BT)], idx_v)
        for j in range(2):
            pltpu.async_copy(tab.at[idx_v.at[j]], gbufs[j][0], gbufs[j][1])

        def step(i, carry):
            l0 = _NB * i
            for j in range(_NB):
                l = l0 + j
                gbuf, gsem = gbufs[j]
                nbuf, nsem = gbufs[(j + 2) % _NB]
                tb, tsem = tbufs[j]

                @pl.when(l + 2 < _L)
                def _():
                    pltpu.async_copy(tab.at[idx_v.at[l + 2]], nbuf, nsem)

                @pl.when(l >= _NB)
                def _():
                    for dt in range(_D // 8):
                        pltpu.make_async_copy(
                            tb.at[pl.ds(dt * 8, 8), pl.ds(0, _BT)],
                            out.at[l, dt, wid], tsem).wait()
                pltpu.make_async_copy(tab.at[idx_v.at[l]], gbuf, gsem).wait()
                _transpose_128x32(gbuf, tb)
                for dt in range(_D // 8):
                    pltpu.async_copy(tb.at[pl.ds(dt * 8, 8), pl.ds(0, _BT)],
                                     out.at[l, dt, wid], tsem)
            return carry

        lax.fori_loop(0, _L // _NB, step, 0)
        for tb, tsem in tbufs:
            for dt in range(_D // 8):
                pltpu.make_async_copy(tb.at[pl.ds(dt * 8, 8), pl.ds(0, _BT)],
                                      out.at[0, dt, wid], tsem).wait()


_gather_call = functools.partial(
    pl.kernel,
    out_type=(jax.ShapeDtypeStruct((_L, _D // 8, _B // _BT, 8, _BT),
                                   jnp.float32),
              jax.ShapeDtypeStruct((_L, _D // 8, _B // _BT, 8, _BT),
                                   jnp.float32)),
    mesh=plsc.VectorSubcoreMesh(core_axis_name="c", subcore_axis_name="s",
                                num_cores=_NC, num_subcores=_NS),
    scratch_types=(
        [pltpu.VMEM((_L, _BT), jnp.int32)]
        + [pltpu.VMEM((_BT, _D), jnp.float32)] * _NB
        + [pltpu.VMEM((_D, _TP), jnp.float32)] * _NB
        + [pltpu.SemaphoreType.DMA] * (2 * _NB)
    ),
    compiler_params=pltpu.CompilerParams(use_tc_tiling_on_sc=False,
                                         needs_layout_passes=False),
)(_gather_body)


def kernel(area, shot, player, W_area, b_area, shot_table, player_table):
    shot_i = shot.astype(jnp.int32)
    player_i = player.astype(jnp.int32)
    emb_area_t = _area_call(area[:, :, 0].T, area[:, :, 1].T,
                            W_area.T, b_area[:, None])
    emb_area = emb_area_t.transpose(2, 0, 1)
    out5_s, out5_p = _gather_call(shot_i.T, player_i.T,
                                  shot_table, player_table)
    emb_shot = out5_s.transpose(2, 4, 0, 1, 3).reshape(_B, _L, _D)
    emb_player = out5_p.transpose(2, 4, 0, 1, 3).reshape(_B, _L, _D)
    return (emb_area, emb_shot, emb_player)


# speedup vs baseline: 1.1146x; 1.1146x over previous
import functools

import jax
import jax.numpy as jnp
from jax import lax
from jax.experimental import pallas as pl
from jax.experimental.pallas import tpu as pltpu
from jax.experimental.pallas import tpu_sc as plsc

_B, _L = 4096, 200
_D = 32
_N = _B * _L

_NC, _NS = 2, 16
_NW = _NC * _NS
_RW = _B // _NW
_CROWS = 16


def _area_body(x0_ref, x1_ref, wt_ref, bt_ref, o_ref):
    x0 = x0_ref[...]
    x1 = x1_ref[...]
    wt = wt_ref[...]
    bt = bt_ref[...]
    w0 = wt[:, 0:1][None]
    w1 = wt[:, 1:2][None]
    y = x0[:, None, :] * w0 + x1[:, None, :] * w1 + bt[None]
    o_ref[...] = jnp.maximum(y, 0.0)


_AREA_LB = 8

_area_call = pl.pallas_call(
    _area_body,
    out_shape=jax.ShapeDtypeStruct((_L, _D, _B), jnp.float32),
    grid=(_L // _AREA_LB,),
    in_specs=[
        pl.BlockSpec((_AREA_LB, _B), lambda i: (i, 0)),
        pl.BlockSpec((_AREA_LB, _B), lambda i: (i, 0)),
        pl.BlockSpec((_D, 2), lambda i: (0, 0)),
        pl.BlockSpec((_D, 1), lambda i: (0, 0)),
    ],
    out_specs=pl.BlockSpec((_AREA_LB, _D, _B), lambda i: (i, 0, 0)),
)


_BT = 128
_LANES = 16


_TP = _BT + 1


def _transpose_128x32(grows, trows):
    rows0 = lax.iota(jnp.int32, _LANES)
    rows1 = rows0 + _LANES
    for b0 in range(0, _BT, 4):
        vals = []
        for b in range(b0, b0 + 4):
            col = jnp.full((_LANES,), b, jnp.int32)
            for dh, rows in ((0, rows0), (1, rows1)):
                x = grows[b, pl.ds(dh * _LANES, _LANES)]
                vals.append((rows, col, x))
        for rows, col, x in vals:
            plsc.store_scatter(trows, [rows, col], x)


def _gather_body(shotT_hbm, playerT_hbm, stab_hbm, ptab_hbm, out_s, out_p,
                 idx_v, g0, g1, trows, trows1, sem0, sem1, wsem, wsem1):
    wid = lax.axis_index("s") * _NC + lax.axis_index("c")
    for tab, idxT_hbm, out in ((stab_hbm, shotT_hbm, out_s),
                               (ptab_hbm, playerT_hbm, out_p)):
        pltpu.sync_copy(idxT_hbm.at[:, pl.ds(wid * _BT, _BT)], idx_v)
        pltpu.async_copy(tab.at[idx_v.at[0]], g0, sem0)

        def step(i, carry):
            l0 = 2 * i

            def do_l(l, gbuf, gsem, nbuf, nsem, issue_next, tb, tsem):
                @pl.when(issue_next)
                def _():
                    pltpu.async_copy(tab.at[idx_v.at[l + 1]], nbuf, nsem)

                @pl.when(l >= 2)
                def _():
                    for dt in range(_D // 8):
                        pltpu.make_async_copy(
                            tb.at[pl.ds(dt * 8, 8), pl.ds(0, _BT)],
                            out.at[l, dt, wid], tsem).wait()
                pltpu.make_async_copy(tab.at[idx_v.at[l]], gbuf, gsem).wait()
                _transpose_128x32(gbuf, tb)
                for dt in range(_D // 8):
                    pltpu.async_copy(tb.at[pl.ds(dt * 8, 8), pl.ds(0, _BT)],
                                     out.at[l, dt, wid], tsem)

            do_l(l0, g0, sem0, g1, sem1, i < _L // 2, trows, wsem)
            do_l(l0 + 1, g1, sem1, g0, sem0, i < _L // 2 - 1, trows1, wsem1)
            return carry

        lax.fori_loop(0, _L // 2, step, 0)
        for tb, tsem in ((trows, wsem), (trows1, wsem1)):
            for dt in range(_D // 8):
                pltpu.make_async_copy(tb.at[pl.ds(dt * 8, 8), pl.ds(0, _BT)],
                                      out.at[0, dt, wid], tsem).wait()


_gather_call = functools.partial(
    pl.kernel,
    out_type=(jax.ShapeDtypeStruct((_L, _D // 8, _B // _BT, 8, _BT),
                                   jnp.float32),
              jax.ShapeDtypeStruct((_L, _D // 8, _B // _BT, 8, _BT),
                                   jnp.float32)),
    mesh=plsc.VectorSubcoreMesh(core_axis_name="c", subcore_axis_name="s",
                                num_cores=_NC, num_subcores=_NS),
    scratch_types=[
        pltpu.VMEM((_L, _BT), jnp.int32),
        pltpu.VMEM((_BT, _D), jnp.float32),
        pltpu.VMEM((_BT, _D), jnp.float32),
        pltpu.VMEM((_D, _TP), jnp.float32),
        pltpu.VMEM((_D, _TP), jnp.float32),
        pltpu.SemaphoreType.DMA,
        pltpu.SemaphoreType.DMA,
        pltpu.SemaphoreType.DMA,
        pltpu.SemaphoreType.DMA,
    ],
    compiler_params=pltpu.CompilerParams(use_tc_tiling_on_sc=False,
                                         needs_layout_passes=False),
)(_gather_body)


def kernel(area, shot, player, W_area, b_area, shot_table, player_table):
    shot_i = shot.astype(jnp.int32)
    player_i = player.astype(jnp.int32)
    emb_area_t = _area_call(area[:, :, 0].T, area[:, :, 1].T,
                            W_area.T, b_area[:, None])
    emb_area = emb_area_t.transpose(2, 0, 1)
    out5_s, out5_p = _gather_call(shot_i.T, player_i.T,
                                  shot_table, player_table)
    emb_shot = out5_s.transpose(2, 4, 0, 1, 3).reshape(_B, _L, _D)
    emb_player = out5_p.transpose(2, 4, 0, 1, 3).reshape(_B, _L, _D)
    return (emb_area, emb_shot, emb_player)
